# baseline (speedup 1.0000x reference)
import functools

import jax
import jax.numpy as jnp
from jax import lax
from jax.experimental import pallas as pl
from jax.experimental.pallas import tpu as pltpu
from jax.experimental.pallas import tpu_sc as plsc

NC = 2
NS = 16
NWK = NC * NS
LN = 16


@functools.partial(jax.jit, static_argnums=(2, 3, 4))
def _deg_pass(dst, etype, N, R, E):
    RN = R * N
    rpt = (-(-RN // NS) + 127) & ~127
    RNP = NS * rpt
    ept = E // NWK
    K = 512
    chunks = ept // K
    mesh = plsc.VectorSubcoreMesh(core_axis_name="c", subcore_axis_name="s")

    @functools.partial(
        pl.kernel,
        out_type=jax.ShapeDtypeStruct((NC, RNP), jnp.float32),
        mesh=mesh,
        scratch_types=[
            pltpu.VMEM_SHARED((RNP,), jnp.float32),
            pltpu.VMEM((K,), jnp.int32),
            pltpu.VMEM((K,), jnp.int32),
            pltpu.VMEM((K,), jnp.int32),
            pltpu.VMEM((K,), jnp.float32),
        ],
    )
    def body(dst_hbm, et_hbm, degp_hbm, acc, dbuf, tbuf, fbuf, ob):
        c = lax.axis_index("c")
        s = lax.axis_index("s")
        wid = c * NS + s

        zf = jnp.zeros((LN,), jnp.float32)
        onef = jnp.ones((LN,), jnp.float32)

        def zrow(i, carry):
            ob[pl.ds(i * LN, LN)] = zf
            return carry

        lax.fori_loop(0, K // LN, zrow, 0)

        for j in range(rpt // K):
            pltpu.sync_copy(ob, acc.at[pl.ds(s * rpt + j * K, K)])
        if rpt % K:
            pltpu.sync_copy(ob.at[pl.ds(0, rpt % K)],
                            acc.at[pl.ds(s * rpt + (rpt // K) * K, rpt % K)])
        plsc.subcore_barrier()

        def orow(i, carry):
            ob[pl.ds(i * LN, LN)] = onef
            return carry

        lax.fori_loop(0, K // LN, orow, 0)

        def chunk(ci, carry):
            base = wid * ept + ci * K
            pltpu.sync_copy(dst_hbm.at[pl.ds(base, K)], dbuf)
            pltpu.sync_copy(et_hbm.at[pl.ds(base, K)], tbuf)

            def vec(v, cc):
                sl = pl.ds(v * LN, LN)
                fbuf[sl] = tbuf[sl] * N + dbuf[sl]
                return cc

            lax.fori_loop(0, K // LN, vec, 0)
            pltpu.sync_copy(ob, acc.at[fbuf], add=True)
            return carry

        lax.fori_loop(0, chunks, chunk, 0)
        plsc.subcore_barrier()
        pltpu.sync_copy(acc.at[pl.ds(s * rpt, rpt)],
                        degp_hbm.at[c, pl.ds(s * rpt, rpt)])

    return body(dst, etype).reshape(NC * RNP)


@functools.partial(jax.jit, static_argnums=(5, 6, 7, 8, 9))
def _weight_pass(degf, dst, etype, c1f, c2f, N, R, NB, E, RNP):
    ept = E // NWK
    KW = 2048
    chunks = ept // KW
    mesh = plsc.VectorSubcoreMesh(core_axis_name="c", subcore_axis_name="s")

    @functools.partial(
        pl.kernel,
        out_type=tuple(jax.ShapeDtypeStruct((E,), jnp.float32)
                       for _ in range(1 + 2 * NB)),
        mesh=mesh,
        scratch_types=[
            pltpu.VMEM((KW,), jnp.int32),
            pltpu.VMEM((KW,), jnp.int32),
            pltpu.VMEM((KW,), jnp.int32),
            pltpu.VMEM((KW,), jnp.int32),
            pltpu.VMEM((KW,), jnp.float32),
            pltpu.VMEM((KW,), jnp.float32),
            pltpu.VMEM((KW,), jnp.float32),
            pltpu.VMEM((KW,), jnp.float32),
            pltpu.VMEM((KW,), jnp.float32),
            pltpu.VMEM((KW,), jnp.float32),
            pltpu.VMEM((KW,), jnp.float32),
            pltpu.VMEM((R * NB + LN,), jnp.float32),
            pltpu.VMEM((R * NB + LN,), jnp.float32),
            pltpu.SemaphoreType.DMA,
        ],
    )
    def body(degf_hbm, dst_hbm, et_hbm, c1_hbm, c2_hbm,
             w0_hbm, w10_hbm, w11_hbm, w20_hbm, w21_hbm,
             dbuf, tbuf, f0buf, f1buf, d0buf, d1buf,
             o0, o10, o11, o20, o21, c1v, c2v, sem):
        c = lax.axis_index("c")
        s = lax.axis_index("s")
        wid = c * NS + s

        one = jnp.ones((LN,), jnp.float32)
        zero = jnp.zeros((LN,), jnp.float32)

        pltpu.sync_copy(c1_hbm, c1v.at[pl.ds(0, R * NB)])
        pltpu.sync_copy(c2_hbm, c2v.at[pl.ds(0, R * NB)])
        cb1 = [[jnp.full((LN,), c1v[pl.ds(r * NB + b, LN)][0], jnp.float32)
                for b in range(NB)] for r in range(R)]
        cb2 = [[jnp.full((LN,), c2v[pl.ds(r * NB + b, LN)][0], jnp.float32)
                for b in range(NB)] for r in range(R)]

        def sel(tt, tab, b):
            v = tab[R - 1][b]
            for r in range(R - 2, -1, -1):
                v = jnp.where(tt == r, tab[r][b], v)
            return v

        def chunk(ci, carry):
            base = wid * ept + ci * KW
            pltpu.sync_copy(dst_hbm.at[pl.ds(base, KW)], dbuf)
            pltpu.sync_copy(et_hbm.at[pl.ds(base, KW)], tbuf)

            def fvec(v, cc):
                sl = pl.ds(v * LN, LN)
                f = tbuf[sl] * N + dbuf[sl]
                f0buf[sl] = f
                f1buf[sl] = f + RNP
                return cc

            lax.fori_loop(0, KW // LN, fvec, 0)
            pltpu.async_copy(degf_hbm.at[f0buf], d0buf, sem).wait()
            pltpu.async_copy(degf_hbm.at[f1buf], d1buf, sem).wait()

            def vec(v, cc):
                sl = pl.ds(v * LN, LN)
                d = d0buf[sl] + d1buf[sl]
                r = one / jnp.maximum(d, one)
                nw = jnp.where(d > zero, r, zero)
                tt = tbuf[sl]
                nw = jnp.where(tt < R, nw, zero)
                o0[sl] = nw
                o10[sl] = nw * sel(tt, cb1, 0)
                o11[sl] = nw * sel(tt, cb1, 1)
                o20[sl] = nw * sel(tt, cb2, 0)
                o21[sl] = nw * sel(tt, cb2, 1)
                return cc

            lax.fori_loop(0, KW // LN, vec, 0)
            esl = pl.ds(base, KW)
            pltpu.sync_copy(o0, w0_hbm.at[esl])
            pltpu.sync_copy(o10, w10_hbm.at[esl])
            pltpu.sync_copy(o11, w11_hbm.at[esl])
            pltpu.sync_copy(o20, w20_hbm.at[esl])
            pltpu.sync_copy(o21, w21_hbm.at[esl])
            return carry

        lax.fori_loop(0, chunks, chunk, 0)

    return body(degf, dst, etype, c1f, c2f)


@functools.partial(jax.jit, static_argnums=(5, 6, 7, 8, 9, 10, 11))
def _layer_pass(x, src, dst, wA, wB, N, E, Din, Dout, Eff, NBW, K):
    ept = E // NWK
    chunks = ept // K
    rpt = (N // NS) & ~7
    tail = N - NS * rpt
    use_orows = Din != Dout
    gE = Eff // LN
    mesh = plsc.VectorSubcoreMesh(core_axis_name="c", subcore_axis_name="s")

    scratch = [
        pltpu.VMEM_SHARED((N, Dout), jnp.float32),
        pltpu.VMEM((K,), jnp.int32),
        pltpu.VMEM((K,), jnp.int32),
        pltpu.VMEM((K + LN,), jnp.float32),
        pltpu.VMEM((K + LN,), jnp.float32),
        pltpu.VMEM((K, Din), jnp.float32),
    ]
    if use_orows:
        scratch.append(pltpu.VMEM((K, Dout), jnp.float32))
    scratch.append(pltpu.SemaphoreType.DMA)

    @functools.partial(
        pl.kernel,
        out_type=jax.ShapeDtypeStruct((NC, N, Dout), jnp.float32),
        mesh=mesh,
        scratch_types=scratch,
    )
    def body(x_hbm, src_hbm, dst_hbm, wA_hbm, wB_hbm, out_hbm, *scr):
        if use_orows:
            acc, sbuf, dbuf, wb0, wb1, rows, orows, sem = scr
        else:
            acc, sbuf, dbuf, wb0, wb1, rows, sem = scr
            orows = rows
        c = lax.axis_index("c")
        s = lax.axis_index("s")
        wid = c * NS + s

        zf = jnp.zeros((LN,), jnp.float32)

        def zrow(i, carry):
            for j in range(Dout // LN):
                orows[i, pl.ds(j * LN, LN)] = zf
            return carry

        lax.fori_loop(0, K, zrow, 0)

        nfull, rem = rpt // K, rpt % K
        for j in range(nfull):
            pltpu.sync_copy(orows.at[pl.ds(0, K), :] if use_orows else
                            rows.at[pl.ds(0, K), pl.ds(0, Dout)],
                            acc.at[pl.ds(s * rpt + j * K, K)])
        if rem:
            pltpu.sync_copy(orows.at[pl.ds(0, rem), :] if use_orows else
                            rows.at[pl.ds(0, rem), pl.ds(0, Dout)],
                            acc.at[pl.ds(s * rpt + nfull * K, rem)])
        if tail:
            @pl.when(s == 0)
            def _():
                pltpu.sync_copy(orows.at[pl.ds(0, tail), :] if use_orows else
                                rows.at[pl.ds(0, tail), pl.ds(0, Dout)],
                                acc.at[pl.ds(NS * rpt, tail)])
        plsc.subcore_barrier()

        def chunk(ci, carry):
            base = wid * ept + ci * K
            pltpu.sync_copy(src_hbm.at[pl.ds(base, K)], sbuf)
            pltpu.sync_copy(dst_hbm.at[pl.ds(base, K)], dbuf)
            pltpu.sync_copy(wA_hbm.at[pl.ds(base, K)], wb0.at[pl.ds(0, K)])
            if NBW == 2:
                pltpu.sync_copy(wB_hbm.at[pl.ds(base, K)], wb1.at[pl.ds(0, K)])
            pltpu.async_copy(x_hbm.at[sbuf], rows, sem).wait()

            def rowfn(k, cc):
                w0v = jnp.full((LN,), wb0[pl.ds(k, LN)][0], jnp.float32)
                if NBW == 2:
                    w1v = jnp.full((LN,), wb1[pl.ds(k, LN)][0], jnp.float32)
                    for j in range(gE):
                        a = rows[k, pl.ds(j * LN, LN)]
                        b = rows[k, pl.ds(Eff + j * LN, LN)]
                        orows[k, pl.ds(j * LN, LN)] = a * w0v + b * w1v
                    if not use_orows:
                        for j in range(gE, 2 * gE):
                            rows[k, pl.ds(j * LN, LN)] = zf
                else:
                    for j in range(gE):
                        sl = pl.ds(j * LN, LN)
                        rows[k, sl] = rows[k, sl] * w0v
                return cc

            lax.fori_loop(0, K, rowfn, 0)
            pltpu.sync_copy(orows if use_orows else rows, acc.at[dbuf], add=True)
            return carry

        lax.fori_loop(0, chunks, chunk, 0)
        plsc.subcore_barrier()
        pltpu.sync_copy(acc.at[pl.ds(s * rpt, rpt)],
                        out_hbm.at[c, pl.ds(s * rpt, rpt)])
        if tail:
            @pl.when(s == 0)
            def _():
                pltpu.sync_copy(acc.at[pl.ds(NS * rpt, tail)],
                                out_hbm.at[c, pl.ds(NS * rpt, tail)])

    return body(x, src, dst, wA, wB)


def _tc_relu_mm(p, bcat):
    _, N, H = p.shape
    DO = bcat.shape[1]
    BR = 1000
    grid = (N // BR,)

    def body(p_ref, b_ref, y_ref):
        h = jnp.maximum(p_ref[0] + p_ref[1], 0.0)
        y_ref[...] = jnp.dot(h, b_ref[...], preferred_element_type=jnp.float32)

    return pl.pallas_call(
        body,
        grid=grid,
        in_specs=[
            pl.BlockSpec((NC, BR, H), lambda i: (0, i, 0)),
            pl.BlockSpec((H, DO), lambda i: (0, 0)),
        ],
        out_specs=pl.BlockSpec((BR, DO), lambda i: (i, 0)),
        out_shape=jax.ShapeDtypeStruct((N, DO), jnp.float32),
    )(p, bcat)


def _tc_add(p, DO):
    _, N, D = p.shape
    BR = 2000
    grid = (N // BR,)

    def body(p_ref, y_ref):
        y_ref[...] = p_ref[0, :, :DO] + p_ref[1, :, :DO]

    return pl.pallas_call(
        body,
        grid=grid,
        in_specs=[pl.BlockSpec((NC, BR, D), lambda i: (0, i, 0))],
        out_specs=pl.BlockSpec((BR, DO), lambda i: (i, 0)),
        out_shape=jax.ShapeDtypeStruct((N, DO), jnp.float32),
    )(p)


def kernel(embed, basis1, coeff1, basis2, coeff2, edge_index, edge_type):
    N, H = embed.shape
    NB = basis1.shape[0]
    OUT = basis2.shape[2]
    R = coeff1.shape[0]
    E = edge_type.shape[0]

    eptp = -(-E // NWK)
    eptp = -(-eptp // 256) * 256
    EP = NWK * eptp
    pad = EP - E
    src = jnp.concatenate([edge_index[0], jnp.zeros((pad,), jnp.int32)])
    dst = jnp.concatenate([edge_index[1], jnp.zeros((pad,), jnp.int32)])
    etp = jnp.concatenate([edge_type, jnp.full((pad,), R, jnp.int32)])

    rpt_d = (-(-(R * N) // NS) + 127) & ~127
    RNP = NS * rpt_d
    degf = _deg_pass(dst, etp, N, R, EP)
    w0, w10, w11, w20, w21 = _weight_pass(degf, dst, etp,
                                          coeff1.reshape(-1),
                                          coeff2.reshape(-1), N, R, NB, EP, RNP)

    p0 = _layer_pass(embed, src, dst, w0, w0, N, EP, H, H, H, 1, 256)
    bcat1 = basis1.transpose(1, 0, 2).reshape(H, NB * H)
    y1 = _tc_relu_mm(p0, bcat1)
    p1 = _layer_pass(y1, src, dst, w10, w11, N, EP, NB * H, H, H, 2, 128)
    bcat2 = basis2.transpose(1, 0, 2).reshape(H, NB * OUT)
    bcat2p = jnp.concatenate(
        [bcat2, jnp.zeros((H, H - NB * OUT), jnp.float32)], axis=1)
    y2 = _tc_relu_mm(p1, bcat2p)
    p2 = _layer_pass(y2, src, dst, w20, w21, N, EP, H, H, OUT, 2, 256)

    return _tc_add(p2, OUT)


# speedup vs baseline: 1.9838x; 1.9838x over previous
import functools

import jax
import jax.numpy as jnp
from jax import lax
from jax.experimental import pallas as pl
from jax.experimental.pallas import tpu as pltpu
from jax.experimental.pallas import tpu_sc as plsc

NC = 2
NS = 16
NWK = NC * NS
LN = 16


@functools.partial(jax.jit, static_argnums=(2, 3, 4))
def _deg_pass(dst, etype, N, R, E):
    RN = R * N
    rpt = (-(-RN // NS) + 127) & ~127
    RNP = NS * rpt
    ept = E // NWK
    K = 512
    chunks = ept // K
    mesh = plsc.VectorSubcoreMesh(core_axis_name="c", subcore_axis_name="s")

    @functools.partial(
        pl.kernel,
        out_type=jax.ShapeDtypeStruct((NC, RNP), jnp.float32),
        mesh=mesh,
        scratch_types=[
            pltpu.VMEM_SHARED((RNP,), jnp.float32),
            pltpu.VMEM((K,), jnp.int32),
            pltpu.VMEM((K,), jnp.int32),
            pltpu.VMEM((K,), jnp.int32),
            pltpu.VMEM((K,), jnp.float32),
        ],
    )
    def body(dst_hbm, et_hbm, degp_hbm, acc, dbuf, tbuf, fbuf, ob):
        c = lax.axis_index("c")
        s = lax.axis_index("s")
        wid = c * NS + s

        zf = jnp.zeros((LN,), jnp.float32)
        onef = jnp.ones((LN,), jnp.float32)

        def zrow(i, carry):
            ob[pl.ds(i * LN, LN)] = zf
            return carry

        lax.fori_loop(0, K // LN, zrow, 0)

        for j in range(rpt // K):
            pltpu.sync_copy(ob, acc.at[pl.ds(s * rpt + j * K, K)])
        if rpt % K:
            pltpu.sync_copy(ob.at[pl.ds(0, rpt % K)],
                            acc.at[pl.ds(s * rpt + (rpt // K) * K, rpt % K)])
        plsc.subcore_barrier()

        def orow(i, carry):
            ob[pl.ds(i * LN, LN)] = onef
            return carry

        lax.fori_loop(0, K // LN, orow, 0)

        def chunk(ci, carry):
            base = wid * ept + ci * K
            pltpu.sync_copy(dst_hbm.at[pl.ds(base, K)], dbuf)
            pltpu.sync_copy(et_hbm.at[pl.ds(base, K)], tbuf)

            def vec(v, cc):
                sl = pl.ds(v * LN, LN)
                fbuf[sl] = tbuf[sl] * N + dbuf[sl]
                return cc

            lax.fori_loop(0, K // LN, vec, 0)
            pltpu.sync_copy(ob, acc.at[fbuf], add=True)
            return carry

        lax.fori_loop(0, chunks, chunk, 0)
        plsc.subcore_barrier()
        pltpu.sync_copy(acc.at[pl.ds(s * rpt, rpt)],
                        degp_hbm.at[c, pl.ds(s * rpt, rpt)])

    return body(dst, etype).reshape(NC * RNP)


@functools.partial(jax.jit, static_argnums=(5, 6, 7, 8, 9))
def _weight_pass(degf, dst, etype, c1f, c2f, N, R, NB, E, RNP):
    ept = E // NWK
    KW = 2048
    chunks = ept // KW
    mesh = plsc.VectorSubcoreMesh(core_axis_name="c", subcore_axis_name="s")

    @functools.partial(
        pl.kernel,
        out_type=tuple(jax.ShapeDtypeStruct((E,), jnp.float32)
                       for _ in range(1 + 2 * NB)),
        mesh=mesh,
        scratch_types=[
            pltpu.VMEM((KW,), jnp.int32),
            pltpu.VMEM((KW,), jnp.int32),
            pltpu.VMEM((KW,), jnp.int32),
            pltpu.VMEM((KW,), jnp.int32),
            pltpu.VMEM((KW,), jnp.float32),
            pltpu.VMEM((KW,), jnp.float32),
            pltpu.VMEM((KW,), jnp.float32),
            pltpu.VMEM((KW,), jnp.float32),
            pltpu.VMEM((KW,), jnp.float32),
            pltpu.VMEM((KW,), jnp.float32),
            pltpu.VMEM((KW,), jnp.float32),
            pltpu.VMEM((R * NB + LN,), jnp.float32),
            pltpu.VMEM((R * NB + LN,), jnp.float32),
            pltpu.SemaphoreType.DMA,
        ],
    )
    def body(degf_hbm, dst_hbm, et_hbm, c1_hbm, c2_hbm,
             w0_hbm, w10_hbm, w11_hbm, w20_hbm, w21_hbm,
             dbuf, tbuf, f0buf, f1buf, d0buf, d1buf,
             o0, o10, o11, o20, o21, c1v, c2v, sem):
        c = lax.axis_index("c")
        s = lax.axis_index("s")
        wid = c * NS + s

        one = jnp.ones((LN,), jnp.float32)
        zero = jnp.zeros((LN,), jnp.float32)

        pltpu.sync_copy(c1_hbm, c1v.at[pl.ds(0, R * NB)])
        pltpu.sync_copy(c2_hbm, c2v.at[pl.ds(0, R * NB)])
        cb1 = [[jnp.full((LN,), c1v[pl.ds(r * NB + b, LN)][0], jnp.float32)
                for b in range(NB)] for r in range(R)]
        cb2 = [[jnp.full((LN,), c2v[pl.ds(r * NB + b, LN)][0], jnp.float32)
                for b in range(NB)] for r in range(R)]

        def sel(tt, tab, b):
            v = tab[R - 1][b]
            for r in range(R - 2, -1, -1):
                v = jnp.where(tt == r, tab[r][b], v)
            return v

        def chunk(ci, carry):
            base = wid * ept + ci * KW
            pltpu.sync_copy(dst_hbm.at[pl.ds(base, KW)], dbuf)
            pltpu.sync_copy(et_hbm.at[pl.ds(base, KW)], tbuf)

            def fvec(v, cc):
                sl = pl.ds(v * LN, LN)
                f = tbuf[sl] * N + dbuf[sl]
                f0buf[sl] = f
                f1buf[sl] = f + RNP
                return cc

            lax.fori_loop(0, KW // LN, fvec, 0)
            pltpu.async_copy(degf_hbm.at[f0buf], d0buf, sem).wait()
            pltpu.async_copy(degf_hbm.at[f1buf], d1buf, sem).wait()

            def vec(v, cc):
                sl = pl.ds(v * LN, LN)
                d = d0buf[sl] + d1buf[sl]
                r = one / jnp.maximum(d, one)
                nw = jnp.where(d > zero, r, zero)
                tt = tbuf[sl]
                nw = jnp.where(tt < R, nw, zero)
                o0[sl] = nw
                o10[sl] = nw * sel(tt, cb1, 0)
                o11[sl] = nw * sel(tt, cb1, 1)
                o20[sl] = nw * sel(tt, cb2, 0)
                o21[sl] = nw * sel(tt, cb2, 1)
                return cc

            lax.fori_loop(0, KW // LN, vec, 0)
            esl = pl.ds(base, KW)
            pltpu.sync_copy(o0, w0_hbm.at[esl])
            pltpu.sync_copy(o10, w10_hbm.at[esl])
            pltpu.sync_copy(o11, w11_hbm.at[esl])
            pltpu.sync_copy(o20, w20_hbm.at[esl])
            pltpu.sync_copy(o21, w21_hbm.at[esl])
            return carry

        lax.fori_loop(0, chunks, chunk, 0)

    return body(degf, dst, etype, c1f, c2f)


@functools.partial(jax.jit, static_argnums=(5, 6, 7, 8, 9, 10, 11))
def _layer_pass(x, src, dst, wA, wB, N, E, Din, Dout, Eff, NBW, K):
    ept = E // NWK
    chunks = ept // K
    rpt = (N // NS) & ~7
    tail = N - NS * rpt
    gE = Eff // LN
    mesh = plsc.VectorSubcoreMesh(core_axis_name="c", subcore_axis_name="s")

    scratch = [
        pltpu.VMEM_SHARED((N, Dout), jnp.float32),
        pltpu.VMEM((K,), jnp.int32),
        pltpu.VMEM((K,), jnp.int32),
        pltpu.VMEM((K + LN,), jnp.float32),
        pltpu.VMEM((K + LN,), jnp.float32),
        pltpu.VMEM((K, Din), jnp.float32),
    ]
    scratch.append(pltpu.SemaphoreType.DMA)

    @functools.partial(
        pl.kernel,
        out_type=jax.ShapeDtypeStruct((NC, N, Dout), jnp.float32),
        mesh=mesh,
        scratch_types=scratch,
    )
    def body(x_hbm, src_hbm, dst_hbm, wA_hbm, wB_hbm, out_hbm, *scr):
        acc, sbuf, dbuf, wb0, wb1, rows, sem = scr
        c = lax.axis_index("c")
        s = lax.axis_index("s")
        wid = c * NS + s

        zf = jnp.zeros((LN,), jnp.float32)

        def zrow(i, carry):
            for j in range(Dout // LN):
                rows[i, pl.ds(j * LN, LN)] = zf
            return carry

        lax.fori_loop(0, K, zrow, 0)

        nfull, rem = rpt // K, rpt % K
        for j in range(nfull):
            pltpu.sync_copy(rows.at[pl.ds(0, K), pl.ds(0, Dout)],
                            acc.at[pl.ds(s * rpt + j * K, K)])
        if rem:
            pltpu.sync_copy(rows.at[pl.ds(0, rem), pl.ds(0, Dout)],
                            acc.at[pl.ds(s * rpt + nfull * K, rem)])
        if tail:
            @pl.when(s == 0)
            def _():
                pltpu.sync_copy(rows.at[pl.ds(0, tail), pl.ds(0, Dout)],
                                acc.at[pl.ds(NS * rpt, tail)])
        plsc.subcore_barrier()

        def chunk(ci, carry):
            base = wid * ept + ci * K
            pltpu.sync_copy(src_hbm.at[pl.ds(base, K)], sbuf)
            pltpu.sync_copy(dst_hbm.at[pl.ds(base, K)], dbuf)
            pltpu.sync_copy(wA_hbm.at[pl.ds(base, K)], wb0.at[pl.ds(0, K)])
            if NBW == 2:
                pltpu.sync_copy(wB_hbm.at[pl.ds(base, K)], wb1.at[pl.ds(0, K)])
            pltpu.async_copy(x_hbm.at[sbuf], rows, sem).wait()

            def rowfn(k, cc):
                w0v = jnp.full((LN,), wb0[pl.ds(k, LN)][0], jnp.float32)
                if NBW == 2:
                    w1v = jnp.full((LN,), wb1[pl.ds(k, LN)][0], jnp.float32)
                    for j in range(gE):
                        a = rows[k, pl.ds(j * LN, LN)]
                        b = rows[k, pl.ds(Eff + j * LN, LN)]
                        rows[k, pl.ds(j * LN, LN)] = a * w0v + b * w1v
                    for j in range(gE, 2 * gE):
                        rows[k, pl.ds(j * LN, LN)] = zf
                else:
                    for j in range(gE):
                        sl = pl.ds(j * LN, LN)
                        rows[k, sl] = rows[k, sl] * w0v
                return cc

            lax.fori_loop(0, K, rowfn, 0)
            pltpu.sync_copy(rows, acc.at[dbuf], add=True)
            return carry

        lax.fori_loop(0, chunks, chunk, 0)
        plsc.subcore_barrier()
        pltpu.sync_copy(acc.at[pl.ds(s * rpt, rpt)],
                        out_hbm.at[c, pl.ds(s * rpt, rpt)])
        if tail:
            @pl.when(s == 0)
            def _():
                pltpu.sync_copy(acc.at[pl.ds(NS * rpt, tail)],
                                out_hbm.at[c, pl.ds(NS * rpt, tail)])

    return body(x, src, dst, wA, wB)


def _tc_relu_mm(p, bcat):
    _, N, H = p.shape
    DO = bcat.shape[1]
    BR = 1000
    grid = (N // BR,)

    NP = p.shape[0]

    def body(p_ref, b_ref, y_ref):
        h = jnp.maximum(jnp.sum(p_ref[...], axis=0), 0.0)
        y_ref[...] = jnp.dot(h, b_ref[...], preferred_element_type=jnp.float32)

    return pl.pallas_call(
        body,
        grid=grid,
        in_specs=[
            pl.BlockSpec((NP, BR, H), lambda i: (0, i, 0)),
            pl.BlockSpec((H, DO), lambda i: (0, 0)),
        ],
        out_specs=pl.BlockSpec((BR, DO), lambda i: (i, 0)),
        out_shape=jax.ShapeDtypeStruct((N, DO), jnp.float32),
    )(p, bcat)


def _tc_relu_mm2(p, bcat):
    NP, N, H = p.shape
    DO = bcat.shape[1]
    HB = DO // 2
    BR = 1000
    grid = (N // BR,)

    def body(p_ref, b_ref, y_ref):
        h = jnp.maximum(jnp.sum(p_ref[...], axis=0), 0.0)
        y_ref[0] = jnp.dot(h, b_ref[:, :HB], preferred_element_type=jnp.float32)
        y_ref[1] = jnp.dot(h, b_ref[:, HB:], preferred_element_type=jnp.float32)

    return pl.pallas_call(
        body,
        grid=grid,
        in_specs=[
            pl.BlockSpec((NP, BR, H), lambda i: (0, i, 0)),
            pl.BlockSpec((H, DO), lambda i: (0, 0)),
        ],
        out_specs=pl.BlockSpec((2, BR, HB), lambda i: (0, i, 0)),
        out_shape=jax.ShapeDtypeStruct((2, N, HB), jnp.float32),
    )(p, bcat)


def _tc_add(p, DO):
    _, N, D = p.shape
    BR = 2000
    grid = (N // BR,)

    def body(p_ref, y_ref):
        y_ref[...] = p_ref[0, :, :DO] + p_ref[1, :, :DO]

    return pl.pallas_call(
        body,
        grid=grid,
        in_specs=[pl.BlockSpec((NC, BR, D), lambda i: (0, i, 0))],
        out_specs=pl.BlockSpec((BR, DO), lambda i: (i, 0)),
        out_shape=jax.ShapeDtypeStruct((N, DO), jnp.float32),
    )(p)


def kernel(embed, basis1, coeff1, basis2, coeff2, edge_index, edge_type):
    N, H = embed.shape
    NB = basis1.shape[0]
    OUT = basis2.shape[2]
    R = coeff1.shape[0]
    E = edge_type.shape[0]

    eptp = -(-E // NWK)
    eptp = -(-eptp // 256) * 256
    EP = NWK * eptp
    pad = EP - E
    rpt_d0 = (-(-(R * N) // NS) + 127) & ~127
    span = NS * rpt_d0 - R * N
    spread = (jnp.arange(pad, dtype=jnp.int32) * 37) % span
    src = jnp.concatenate([edge_index[0], spread])
    dst = jnp.concatenate([edge_index[1], spread])
    etp = jnp.concatenate([edge_type, jnp.full((pad,), R, jnp.int32)])

    rpt_d = (-(-(R * N) // NS) + 127) & ~127
    RNP = NS * rpt_d
    degf = _deg_pass(dst, etp, N, R, EP)
    w0, w10, w11, w20, w21 = _weight_pass(degf, dst, etp,
                                          coeff1.reshape(-1),
                                          coeff2.reshape(-1), N, R, NB, EP, RNP)

    p0 = _layer_pass(embed, src, dst, w0, w0, N, EP, H, H, H, 1, 256)
    bcat1 = basis1.transpose(1, 0, 2).reshape(H, NB * H)
    y1 = _tc_relu_mm2(p0, bcat1)
    p1a = _layer_pass(y1[0], src, dst, w10, w10, N, EP, H, H, H, 1, 256)
    p1b = _layer_pass(y1[1], src, dst, w11, w11, N, EP, H, H, H, 1, 256)
    p1 = jnp.concatenate([p1a, p1b], axis=0)
    bcat2 = basis2.transpose(1, 0, 2).reshape(H, NB * OUT)
    bcat2p = jnp.concatenate(
        [bcat2, jnp.zeros((H, H - NB * OUT), jnp.float32)], axis=1)
    y2 = _tc_relu_mm(p1, bcat2p)
    p2 = _layer_pass(y2, src, dst, w20, w21, N, EP, H, H, OUT, 2, 256)

    return _tc_add(p2, OUT)


# speedup vs baseline: 2.0899x; 1.0535x over previous
import functools

import jax
import jax.numpy as jnp
from jax import lax
from jax.experimental import pallas as pl
from jax.experimental.pallas import tpu as pltpu
from jax.experimental.pallas import tpu_sc as plsc

NC = 2
NS = 16
NWK = NC * NS
LN = 16


@functools.partial(jax.jit, static_argnums=(2, 3, 4))
def _deg_pass(dst, etype, N, R, E):
    RN = R * N
    rpt = (-(-RN // NS) + 127) & ~127
    RNP = NS * rpt
    ept = E // NWK
    K = 512
    chunks = ept // K
    mesh = plsc.VectorSubcoreMesh(core_axis_name="c", subcore_axis_name="s")

    @functools.partial(
        pl.kernel,
        out_type=jax.ShapeDtypeStruct((NC, RNP), jnp.float32),
        mesh=mesh,
        scratch_types=[
            pltpu.VMEM_SHARED((RNP,), jnp.float32),
            pltpu.VMEM((K,), jnp.int32),
            pltpu.VMEM((K,), jnp.int32),
            pltpu.VMEM((K,), jnp.int32),
            pltpu.VMEM((K,), jnp.float32),
        ],
    )
    def body(dst_hbm, et_hbm, degp_hbm, acc, dbuf, tbuf, fbuf, ob):
        c = lax.axis_index("c")
        s = lax.axis_index("s")
        wid = c * NS + s

        zf = jnp.zeros((LN,), jnp.float32)
        onef = jnp.ones((LN,), jnp.float32)

        def zrow(i, carry):
            ob[pl.ds(i * LN, LN)] = zf
            return carry

        lax.fori_loop(0, K // LN, zrow, 0)

        for j in range(rpt // K):
            pltpu.sync_copy(ob, acc.at[pl.ds(s * rpt + j * K, K)])
        if rpt % K:
            pltpu.sync_copy(ob.at[pl.ds(0, rpt % K)],
                            acc.at[pl.ds(s * rpt + (rpt // K) * K, rpt % K)])
        plsc.subcore_barrier()

        def orow(i, carry):
            ob[pl.ds(i * LN, LN)] = onef
            return carry

        lax.fori_loop(0, K // LN, orow, 0)

        def chunk(ci, carry):
            base = wid * ept + ci * K
            pltpu.sync_copy(dst_hbm.at[pl.ds(base, K)], dbuf)
            pltpu.sync_copy(et_hbm.at[pl.ds(base, K)], tbuf)

            def vec(v, cc):
                sl = pl.ds(v * LN, LN)
                fbuf[sl] = tbuf[sl] * N + dbuf[sl]
                return cc

            lax.fori_loop(0, K // LN, vec, 0)
            pltpu.sync_copy(ob, acc.at[fbuf], add=True)
            return carry

        lax.fori_loop(0, chunks, chunk, 0)
        plsc.subcore_barrier()
        pltpu.sync_copy(acc.at[pl.ds(s * rpt, rpt)],
                        degp_hbm.at[c, pl.ds(s * rpt, rpt)])

    return body(dst, etype).reshape(NC * RNP)


@functools.partial(jax.jit, static_argnums=(5, 6, 7, 8, 9))
def _weight_pass(degf, dst, etype, c1f, c2f, N, R, NB, E, RNP):
    ept = E // NWK
    KW = 2048
    chunks = ept // KW
    mesh = plsc.VectorSubcoreMesh(core_axis_name="c", subcore_axis_name="s")

    @functools.partial(
        pl.kernel,
        out_type=tuple(jax.ShapeDtypeStruct((E,), jnp.float32)
                       for _ in range(1 + 2 * NB)),
        mesh=mesh,
        scratch_types=[
            pltpu.VMEM((KW,), jnp.int32),
            pltpu.VMEM((KW,), jnp.int32),
            pltpu.VMEM((KW,), jnp.int32),
            pltpu.VMEM((KW,), jnp.int32),
            pltpu.VMEM((KW,), jnp.float32),
            pltpu.VMEM((KW,), jnp.float32),
            pltpu.VMEM((KW,), jnp.float32),
            pltpu.VMEM((KW,), jnp.float32),
            pltpu.VMEM((KW,), jnp.float32),
            pltpu.VMEM((KW,), jnp.float32),
            pltpu.VMEM((KW,), jnp.float32),
            pltpu.VMEM((R * NB + LN,), jnp.float32),
            pltpu.VMEM((R * NB + LN,), jnp.float32),
            pltpu.SemaphoreType.DMA,
        ],
    )
    def body(degf_hbm, dst_hbm, et_hbm, c1_hbm, c2_hbm,
             w0_hbm, w10_hbm, w11_hbm, w20_hbm, w21_hbm,
             dbuf, tbuf, f0buf, f1buf, d0buf, d1buf,
             o0, o10, o11, o20, o21, c1v, c2v, sem):
        c = lax.axis_index("c")
        s = lax.axis_index("s")
        wid = c * NS + s

        one = jnp.ones((LN,), jnp.float32)
        zero = jnp.zeros((LN,), jnp.float32)

        pltpu.sync_copy(c1_hbm, c1v.at[pl.ds(0, R * NB)])
        pltpu.sync_copy(c2_hbm, c2v.at[pl.ds(0, R * NB)])
        cb1 = [[jnp.full((LN,), c1v[pl.ds(r * NB + b, LN)][0], jnp.float32)
                for b in range(NB)] for r in range(R)]
        cb2 = [[jnp.full((LN,), c2v[pl.ds(r * NB + b, LN)][0], jnp.float32)
                for b in range(NB)] for r in range(R)]

        def sel(tt, tab, b):
            v = tab[R - 1][b]
            for r in range(R - 2, -1, -1):
                v = jnp.where(tt == r, tab[r][b], v)
            return v

        def chunk(ci, carry):
            base = wid * ept + ci * KW
            pltpu.sync_copy(dst_hbm.at[pl.ds(base, KW)], dbuf)
            pltpu.sync_copy(et_hbm.at[pl.ds(base, KW)], tbuf)

            def fvec(v, cc):
                sl = pl.ds(v * LN, LN)
                f = tbuf[sl] * N + dbuf[sl]
                f0buf[sl] = f
                f1buf[sl] = f + RNP
                return cc

            lax.fori_loop(0, KW // LN, fvec, 0)
            pltpu.async_copy(degf_hbm.at[f0buf], d0buf, sem).wait()
            pltpu.async_copy(degf_hbm.at[f1buf], d1buf, sem).wait()

            def vec(v, cc):
                sl = pl.ds(v * LN, LN)
                d = d0buf[sl] + d1buf[sl]
                r = one / jnp.maximum(d, one)
                nw = jnp.where(d > zero, r, zero)
                tt = tbuf[sl]
                nw = jnp.where(tt < R, nw, zero)
                o0[sl] = nw
                o10[sl] = nw * sel(tt, cb1, 0)
                o11[sl] = nw * sel(tt, cb1, 1)
                o20[sl] = nw * sel(tt, cb2, 0)
                o21[sl] = nw * sel(tt, cb2, 1)
                return cc

            lax.fori_loop(0, KW // LN, vec, 0)
            esl = pl.ds(base, KW)
            pltpu.sync_copy(o0, w0_hbm.at[esl])
            pltpu.sync_copy(o10, w10_hbm.at[esl])
            pltpu.sync_copy(o11, w11_hbm.at[esl])
            pltpu.sync_copy(o20, w20_hbm.at[esl])
            pltpu.sync_copy(o21, w21_hbm.at[esl])
            return carry

        lax.fori_loop(0, chunks, chunk, 0)

    return body(degf, dst, etype, c1f, c2f)


@functools.partial(jax.jit, static_argnums=(5, 6, 7, 8, 9, 10, 11))
def _layer_pass(x, src, dst, wA, wB, N, E, Din, Dout, Eff, NBW, K):
    ept = E // NWK
    chunks = ept // K
    rpt = (N // NS) & ~7
    tail = N - NS * rpt
    gE = Eff // LN
    mesh = plsc.VectorSubcoreMesh(core_axis_name="c", subcore_axis_name="s")

    scratch = [
        pltpu.VMEM_SHARED((N, Dout), jnp.float32),
        pltpu.VMEM((K,), jnp.int32),
        pltpu.VMEM((K,), jnp.int32),
        pltpu.VMEM((K + LN,), jnp.float32),
        pltpu.VMEM((K + LN,), jnp.float32),
        pltpu.VMEM((K, Din), jnp.float32),
    ]
    scratch.append(pltpu.SemaphoreType.DMA)

    @functools.partial(
        pl.kernel,
        out_type=jax.ShapeDtypeStruct((NC, N, Dout), jnp.float32),
        mesh=mesh,
        scratch_types=scratch,
    )
    def body(x_hbm, src_hbm, dst_hbm, wA_hbm, wB_hbm, out_hbm, *scr):
        acc, sbuf, dbuf, wb0, wb1, rows, sem = scr
        c = lax.axis_index("c")
        s = lax.axis_index("s")
        wid = c * NS + s

        zf = jnp.zeros((LN,), jnp.float32)

        def zrow(i, carry):
            for j in range(Dout // LN):
                rows[i, pl.ds(j * LN, LN)] = zf
            return carry

        lax.fori_loop(0, K, zrow, 0)

        nfull, rem = rpt // K, rpt % K
        for j in range(nfull):
            pltpu.sync_copy(rows.at[pl.ds(0, K), pl.ds(0, Dout)],
                            acc.at[pl.ds(s * rpt + j * K, K)])
        if rem:
            pltpu.sync_copy(rows.at[pl.ds(0, rem), pl.ds(0, Dout)],
                            acc.at[pl.ds(s * rpt + nfull * K, rem)])
        if tail:
            @pl.when(s == 0)
            def _():
                pltpu.sync_copy(rows.at[pl.ds(0, tail), pl.ds(0, Dout)],
                                acc.at[pl.ds(NS * rpt, tail)])
        plsc.subcore_barrier()

        def chunk(ci, carry):
            base = wid * ept + ci * K
            pltpu.sync_copy(src_hbm.at[pl.ds(base, K)], sbuf)
            pltpu.sync_copy(dst_hbm.at[pl.ds(base, K)], dbuf)
            pltpu.sync_copy(wA_hbm.at[pl.ds(base, K)], wb0.at[pl.ds(0, K)])
            if NBW == 2:
                pltpu.sync_copy(wB_hbm.at[pl.ds(base, K)], wb1.at[pl.ds(0, K)])
            pltpu.async_copy(x_hbm.at[sbuf], rows, sem).wait()

            def rowfn(k, cc):
                w0v = jnp.full((LN,), wb0[pl.ds(k, LN)][0], jnp.float32)
                if NBW == 2:
                    w1v = jnp.full((LN,), wb1[pl.ds(k, LN)][0], jnp.float32)
                    for j in range(gE):
                        a = rows[k, pl.ds(j * LN, LN)]
                        b = rows[k, pl.ds(Eff + j * LN, LN)]
                        rows[k, pl.ds(j * LN, LN)] = a * w0v + b * w1v
                    for j in range(gE, 2 * gE):
                        rows[k, pl.ds(j * LN, LN)] = zf
                else:
                    for j in range(gE):
                        sl = pl.ds(j * LN, LN)
                        rows[k, sl] = rows[k, sl] * w0v
                return cc

            lax.fori_loop(0, K, rowfn, 0)
            pltpu.sync_copy(rows, acc.at[dbuf], add=True)
            return carry

        lax.fori_loop(0, chunks, chunk, 0)
        plsc.subcore_barrier()
        pltpu.sync_copy(acc.at[pl.ds(s * rpt, rpt)],
                        out_hbm.at[c, pl.ds(s * rpt, rpt)])
        if tail:
            @pl.when(s == 0)
            def _():
                pltpu.sync_copy(acc.at[pl.ds(NS * rpt, tail)],
                                out_hbm.at[c, pl.ds(NS * rpt, tail)])

    return body(x, src, dst, wA, wB)


def _tc_relu_mm(p, bcat):
    _, N, H = p.shape
    DO = bcat.shape[1]
    BR = 1000
    grid = (N // BR,)

    NP = p.shape[0]

    def body(p_ref, b_ref, y_ref):
        h = jnp.maximum(jnp.sum(p_ref[...], axis=0), 0.0)
        y_ref[...] = jnp.dot(h, b_ref[...], preferred_element_type=jnp.float32)

    return pl.pallas_call(
        body,
        grid=grid,
        in_specs=[
            pl.BlockSpec((NP, BR, H), lambda i: (0, i, 0)),
            pl.BlockSpec((H, DO), lambda i: (0, 0)),
        ],
        out_specs=pl.BlockSpec((BR, DO), lambda i: (i, 0)),
        out_shape=jax.ShapeDtypeStruct((N, DO), jnp.float32),
    )(p, bcat)


def _tc_relu_mm2(p, bcat):
    NP, N, H = p.shape
    DO = bcat.shape[1]
    HB = DO // 2
    BR = 1000
    grid = (N // BR,)

    def body(p_ref, b_ref, y_ref):
        h = jnp.maximum(jnp.sum(p_ref[...], axis=0), 0.0)
        y_ref[0] = jnp.dot(h, b_ref[:, :HB], preferred_element_type=jnp.float32)
        y_ref[1] = jnp.dot(h, b_ref[:, HB:], preferred_element_type=jnp.float32)

    return pl.pallas_call(
        body,
        grid=grid,
        in_specs=[
            pl.BlockSpec((NP, BR, H), lambda i: (0, i, 0)),
            pl.BlockSpec((H, DO), lambda i: (0, 0)),
        ],
        out_specs=pl.BlockSpec((2, BR, HB), lambda i: (0, i, 0)),
        out_shape=jax.ShapeDtypeStruct((2, N, HB), jnp.float32),
    )(p, bcat)


def _tc_add(p, DO):
    _, N, D = p.shape
    BR = 2000
    grid = (N // BR,)

    def body(p_ref, y_ref):
        y_ref[...] = p_ref[0, :, :DO] + p_ref[1, :, :DO]

    return pl.pallas_call(
        body,
        grid=grid,
        in_specs=[pl.BlockSpec((NC, BR, D), lambda i: (0, i, 0))],
        out_specs=pl.BlockSpec((BR, DO), lambda i: (i, 0)),
        out_shape=jax.ShapeDtypeStruct((N, DO), jnp.float32),
    )(p)


def kernel(embed, basis1, coeff1, basis2, coeff2, edge_index, edge_type):
    N, H = embed.shape
    NB = basis1.shape[0]
    OUT = basis2.shape[2]
    R = coeff1.shape[0]
    E = edge_type.shape[0]

    eptp = -(-E // NWK)
    eptp = -(-eptp // 320) * 320
    EP = NWK * eptp
    pad = EP - E
    rpt_d0 = (-(-(R * N) // NS) + 127) & ~127
    span = NS * rpt_d0 - R * N
    spread = (jnp.arange(pad, dtype=jnp.int32) * 37) % span
    src = jnp.concatenate([edge_index[0], spread])
    dst = jnp.concatenate([edge_index[1], spread])
    etp = jnp.concatenate([edge_type, jnp.full((pad,), R, jnp.int32)])

    rpt_d = (-(-(R * N) // NS) + 127) & ~127
    RNP = NS * rpt_d
    degf = _deg_pass(dst, etp, N, R, EP)
    w0, w10, w11, w20, w21 = _weight_pass(degf, dst, etp,
                                          coeff1.reshape(-1),
                                          coeff2.reshape(-1), N, R, NB, EP, RNP)

    p0 = _layer_pass(embed, src, dst, w0, w0, N, EP, H, H, H, 1, 320)
    bcat1 = basis1.transpose(1, 0, 2).reshape(H, NB * H)
    y1 = _tc_relu_mm2(p0, bcat1)
    p1a = _layer_pass(y1[0], src, dst, w10, w10, N, EP, H, H, H, 1, 320)
    p1b = _layer_pass(y1[1], src, dst, w11, w11, N, EP, H, H, H, 1, 320)
    p1 = jnp.concatenate([p1a, p1b], axis=0)
    bcat2 = basis2.transpose(1, 0, 2).reshape(H, NB * OUT)
    bcat2p = jnp.concatenate(
        [bcat2, jnp.zeros((H, H - NB * OUT), jnp.float32)], axis=1)
    y2 = _tc_relu_mm(p1, bcat2p)
    p2 = _layer_pass(y2, src, dst, w20, w21, N, EP, H, H, OUT, 2, 320)

    return _tc_add(p2, OUT)
